# Initial kernel scaffold; baseline (speedup 1.0000x reference)
import jax
import jax.numpy as jnp
from jax.experimental import pallas as pl


def kernel(node_feat, edge_index, graph_ids, gcn_emb, gin_emb, gcn_W, gcn_b, mlp_W1, mlp_b1, mlp_W2, mlp_b2, eps, ffnn_W0, ffnn_b0, ffnn_W1, ffnn_b1, fc_W, fc_b):
    raise NotImplementedError("write your pallas kernel here")


# speedup vs baseline: 2.4158x; 2.4158x over previous
import functools

import jax
import jax.numpy as jnp
from jax import lax
from jax.experimental import pallas as pl
from jax.experimental.pallas import tpu as pltpu
from jax.experimental.pallas import tpu_sc as plsc

N = 10000
E = 160000
D = 256
V = 128
G = 16
FF = 512
L = 3

H = D // 2
NTILE = 16
NP = 10240
NPT = NP // NTILE
BN = 256
EPT = E // NTILE
K = 128
NCH = 79
EPAD = NTILE * NCH * K

_f32 = jnp.float32


def _deg_body(edge_hbm, deg_hbm, edges_v, hist_v, red_v, out_v, shared):
    c = lax.axis_index("c")
    s = lax.axis_index("s")

    def zero(i, _):
        hist_v[pl.ds(i * 16, 16)] = jnp.zeros((16,), _f32)
        return 0
    lax.fori_loop(0, NP // 16, zero, 0)

    pltpu.sync_copy(edge_hbm.at[pl.ds(c * E + s * EPT, EPT)], edges_v)

    ones = jnp.ones((16,), _f32)

    def hist(i, _):
        idx = edges_v[pl.ds(i * 16, 16)]
        plsc.addupdate_scatter(hist_v, [idx], ones)
        return 0
    lax.fori_loop(0, EPT // 16, hist, 0)

    pltpu.sync_copy(hist_v, shared.at[s])
    plsc.subcore_barrier()
    pltpu.sync_copy(shared.at[:, pl.ds(s * NPT, NPT)], red_v)

    def red(i, _):
        acc = red_v[0, pl.ds(i * 16, 16)]
        for r in range(1, NTILE):
            acc = acc + red_v[r, pl.ds(i * 16, 16)]
        out_v[pl.ds(i * 16, 16)] = acc
        return 0
    lax.fori_loop(0, NPT // 16, red, 0)

    pltpu.sync_copy(out_v, deg_hbm.at[pl.ds(c * NP + s * NPT, NPT)])


@functools.cache
def _deg_call():
    return functools.partial(
        pl.kernel,
        out_type=jax.ShapeDtypeStruct((2 * NP,), _f32),
        mesh=plsc.VectorSubcoreMesh(core_axis_name="c", subcore_axis_name="s"),
        compiler_params=pltpu.CompilerParams(needs_layout_passes=False),
        scratch_types=[
            pltpu.VMEM((EPT,), jnp.int32),
            pltpu.VMEM((NP,), _f32),
            pltpu.VMEM((NTILE, NPT), _f32),
            pltpu.VMEM((NPT,), _f32),
            pltpu.VMEM_SHARED((NTILE, NP), _f32),
        ],
    )(_deg_body)


def _prop_body(table_hbm, src_hbm, dst_hbm, out_hbm, sidx_v, didx_v, rows_v,
               acc_sh):
    c = lax.axis_index("c")
    s = lax.axis_index("s")

    pltpu.sync_copy(src_hbm.at[s], sidx_v)
    pltpu.sync_copy(dst_hbm.at[s], didx_v)

    off = c * NP

    def adj(k, _):
        i = k // (K // 16)
        j = k % (K // 16)
        sidx_v[i, pl.ds(j * 16, 16)] = sidx_v[i, pl.ds(j * 16, 16)] + off
        return 0
    lax.fori_loop(0, NCH * (K // 16), adj, 0)

    def zero(k, _):
        rows_v[k // (K // 16), pl.ds((k % (K // 16)) * 16, 16)] = (
            jnp.zeros((16,), _f32))
        return 0
    lax.fori_loop(0, K * (K // 16), zero, 0)

    for j in range(NPT // K):
        pltpu.sync_copy(rows_v, acc_sh.at[pl.ds(s * NPT + j * K, K)])
    plsc.subcore_barrier()

    def step(i, _):
        pltpu.sync_copy(table_hbm.at[sidx_v.at[i]], rows_v)
        pltpu.sync_copy(rows_v, acc_sh.at[didx_v.at[i]], add=True)
        return 0
    lax.fori_loop(0, NCH, step, 0)

    plsc.subcore_barrier()
    pltpu.sync_copy(acc_sh.at[pl.ds(s * NPT, NPT)],
                    out_hbm.at[c, pl.ds(s * NPT, NPT)])


@functools.cache
def _prop_call():
    return functools.partial(
        pl.kernel,
        out_type=jax.ShapeDtypeStruct((2, NP, H), _f32),
        mesh=plsc.VectorSubcoreMesh(core_axis_name="c", subcore_axis_name="s"),
        compiler_params=pltpu.CompilerParams(needs_layout_passes=False),
        scratch_types=[
            pltpu.VMEM((NCH, K), jnp.int32),
            pltpu.VMEM((NCH, K), jnp.int32),
            pltpu.VMEM((K, H), _f32),
            pltpu.VMEM_SHARED((NP, H), _f32),
        ],
    )(_prop_body)


NFCH = NPT // K


def _embg_body(tab_hbm, feat_hbm, gcn_hbm, gin_hbm, fidx_v, gidx_v, rows_v):
    c = lax.axis_index("c")
    s = lax.axis_index("s")

    pltpu.sync_copy(feat_hbm.at[s], fidx_v)

    for which in range(2):
        off = c * V + which * (2 * V)
        out = gcn_hbm if which == 0 else gin_hbm

        def adj(k, _):
            i = k // (K // 16)
            j = k % (K // 16)
            gidx_v[i, pl.ds(j * 16, 16)] = fidx_v[i, pl.ds(j * 16, 16)] + off
            return 0
        lax.fori_loop(0, NFCH * (K // 16), adj, 0)

        for i in range(NFCH):
            pltpu.sync_copy(tab_hbm.at[gidx_v.at[i]], rows_v)
            pltpu.sync_copy(rows_v, out.at[c, pl.ds(s * NPT + i * K, K)])


@functools.cache
def _embg_call():
    return functools.partial(
        pl.kernel,
        out_type=[jax.ShapeDtypeStruct((2, NP, H), _f32),
                  jax.ShapeDtypeStruct((2, NP, H), _f32)],
        mesh=plsc.VectorSubcoreMesh(core_axis_name="c", subcore_axis_name="s"),
        compiler_params=pltpu.CompilerParams(needs_layout_passes=False),
        scratch_types=[
            pltpu.VMEM((NFCH, K), jnp.int32),
            pltpu.VMEM((NFCH, K), jnp.int32),
            pltpu.VMEM((K, H), _f32),
        ],
    )(_embg_body)


def _norm_body(dego_ref, degi_ref, gcn_ref, gcns_ref, no_ref, ni_ref):
    no = 1.0 / jnp.sqrt(jnp.maximum(dego_ref[...], 1.0))
    ni = 1.0 / jnp.sqrt(jnp.maximum(degi_ref[...], 1.0))
    no_ref[...] = no
    ni_ref[...] = ni
    gcns_ref[0] = gcn_ref[0] * no
    gcns_ref[1] = gcn_ref[1] * no


_norm_call = pl.pallas_call(
    _norm_body,
    grid=(NP // BN,),
    in_specs=[
        pl.BlockSpec((BN, 1), lambda i: (i, 0)),
        pl.BlockSpec((BN, 1), lambda i: (i, 0)),
        pl.BlockSpec((2, BN, H), lambda i: (0, i, 0)),
    ],
    out_specs=[
        pl.BlockSpec((2, BN, H), lambda i: (0, i, 0)),
        pl.BlockSpec((BN, 1), lambda i: (i, 0)),
        pl.BlockSpec((BN, 1), lambda i: (i, 0)),
    ],
    out_shape=[
        jax.ShapeDtypeStruct((2, NP, H), _f32),
        jax.ShapeDtypeStruct((NP, 1), _f32),
        jax.ShapeDtypeStruct((NP, 1), _f32),
    ],
)


def _gcn_body(agg_ref, ni_ref, no_ref, w_ref, b_ref, out_ref, *, scale_out):
    ni = ni_ref[...]
    x = jnp.concatenate([agg_ref[0], agg_ref[1]], axis=1) * ni
    y = jnp.dot(x, w_ref[...], preferred_element_type=_f32) + b_ref[...]
    y = jnp.maximum(y, 0.0)
    if scale_out:
        y = y * no_ref[...]
    out_ref[0] = y[:, :H]
    out_ref[1] = y[:, H:]


def _make_gcn_call(scale_out):
    return pl.pallas_call(
        functools.partial(_gcn_body, scale_out=scale_out),
        grid=(NP // BN,),
        in_specs=[
            pl.BlockSpec((2, BN, H), lambda i: (0, i, 0)),
            pl.BlockSpec((BN, 1), lambda i: (i, 0)),
            pl.BlockSpec((BN, 1), lambda i: (i, 0)),
            pl.BlockSpec((D, D), lambda i: (0, 0)),
            pl.BlockSpec((1, D), lambda i: (0, 0)),
        ],
        out_specs=pl.BlockSpec((2, BN, H), lambda i: (0, i, 0)),
        out_shape=jax.ShapeDtypeStruct((2, NP, H), _f32),
    )


_gcn_mid_call = _make_gcn_call(True)
_gcn_last_call = _make_gcn_call(False)


def _gin_body(h_ref, agg_ref, eps_ref, w1_ref, b1_ref, w2_ref, b2_ref,
              out_ref):
    e = 1.0 + eps_ref[0, 0]
    x = (jnp.concatenate([h_ref[0], h_ref[1]], axis=1) * e
         + jnp.concatenate([agg_ref[0], agg_ref[1]], axis=1))
    t = jnp.dot(x, w1_ref[...], preferred_element_type=_f32) + b1_ref[...]
    t = jnp.maximum(t, 0.0)
    y = jnp.dot(t, w2_ref[...], preferred_element_type=_f32) + b2_ref[...]
    out_ref[0] = y[:, :H]
    out_ref[1] = y[:, H:]


_gin_call = pl.pallas_call(
    _gin_body,
    grid=(NP // BN,),
    in_specs=[
        pl.BlockSpec((2, BN, H), lambda i: (0, i, 0)),
        pl.BlockSpec((2, BN, H), lambda i: (0, i, 0)),
        pl.BlockSpec((1, 1), lambda i: (0, 0)),
        pl.BlockSpec((D, D), lambda i: (0, 0)),
        pl.BlockSpec((1, D), lambda i: (0, 0)),
        pl.BlockSpec((D, D), lambda i: (0, 0)),
        pl.BlockSpec((1, D), lambda i: (0, 0)),
    ],
    out_specs=pl.BlockSpec((2, BN, H), lambda i: (0, i, 0)),
    out_shape=jax.ShapeDtypeStruct((2, NP, H), _f32),
)


def _pool_body(gcn_ref, gin_ref, gid_ref, pool_ref, cnt_ref):
    i = pl.program_id(0)
    gid = gid_ref[...]
    oh = (gid == lax.broadcasted_iota(jnp.int32, (1, G), 1)).astype(_f32)

    dn = (((0,), (0,)), ((), ()))

    @pl.when(i == 0)
    def _():
        pool_ref[...] = jnp.zeros((G, 2 * D), _f32)
        cnt_ref[...] = jnp.zeros((G, 1), _f32)

    pool_ref[:, 0 * H:1 * H] += lax.dot_general(
        oh, gcn_ref[0], dn, preferred_element_type=_f32,
        precision=lax.Precision.HIGHEST)
    pool_ref[:, 1 * H:2 * H] += lax.dot_general(
        oh, gcn_ref[1], dn, preferred_element_type=_f32,
        precision=lax.Precision.HIGHEST)
    pool_ref[:, 2 * H:3 * H] += lax.dot_general(
        oh, gin_ref[0], dn, preferred_element_type=_f32,
        precision=lax.Precision.HIGHEST)
    pool_ref[:, 3 * H:4 * H] += lax.dot_general(
        oh, gin_ref[1], dn, preferred_element_type=_f32,
        precision=lax.Precision.HIGHEST)
    cnt_ref[...] += lax.dot_general(
        oh, jnp.ones((BN, 1), _f32), dn, preferred_element_type=_f32,
        precision=lax.Precision.HIGHEST)


_pool_call = pl.pallas_call(
    _pool_body,
    grid=(NP // BN,),
    in_specs=[
        pl.BlockSpec((2, BN, H), lambda i: (0, i, 0)),
        pl.BlockSpec((2, BN, H), lambda i: (0, i, 0)),
        pl.BlockSpec((BN, 1), lambda i: (i, 0)),
    ],
    out_specs=[
        pl.BlockSpec((G, 2 * D), lambda i: (0, 0)),
        pl.BlockSpec((G, 1), lambda i: (0, 0)),
    ],
    out_shape=[
        jax.ShapeDtypeStruct((G, 2 * D), _f32),
        jax.ShapeDtypeStruct((G, 1), _f32),
    ],
)


def _head_body(pool_ref, cnt_ref, w0_ref, b0_ref, w1_ref, b1_ref,
               fcw_ref, fcb_ref, out_ref):
    cnt = jnp.maximum(cnt_ref[...], 1.0)
    x = pool_ref[...] / cnt
    x = jnp.maximum(jnp.dot(x, w0_ref[...], preferred_element_type=_f32)
                    + b0_ref[...], 0.0)
    x = jnp.maximum(jnp.dot(x, w1_ref[...], preferred_element_type=_f32)
                    + b1_ref[...], 0.0)
    z = jnp.dot(x, fcw_ref[...], preferred_element_type=_f32) + fcb_ref[...]
    out_ref[...] = jax.nn.sigmoid(z)


_head_call = pl.pallas_call(
    _head_body,
    out_shape=jax.ShapeDtypeStruct((G, 1), _f32),
)


def kernel(node_feat, edge_index, graph_ids, gcn_emb, gin_emb, gcn_W, gcn_b,
           mlp_W1, mlp_b1, mlp_W2, mlp_b2, eps,
           ffnn_W0, ffnn_b0, ffnn_W1, ffnn_b1, fc_W, fc_b):
    src = edge_index[0]
    dst = edge_index[1]

    pad = EPAD - E
    perm = jnp.argsort(dst, stable=True)
    src_s = jnp.concatenate([src[perm], jnp.zeros((pad,), jnp.int32)]
                            ).reshape(NTILE, NCH, K)
    dst_s = jnp.concatenate([dst[perm], jnp.full((pad,), NP - 8, jnp.int32)]
                            ).reshape(NTILE, NCH, K)

    feat_r = jnp.concatenate(
        [node_feat, jnp.zeros((NP - N,), jnp.int32)]).reshape(NTILE, NFCH, K)
    gid_c = jnp.concatenate(
        [graph_ids, jnp.full((NP - N,), G, jnp.int32)]).reshape(NP, 1)

    deg = _deg_call()(edge_index.reshape(2 * E))
    dego = deg[:NP].reshape(NP, 1)
    degi = deg[NP:].reshape(NP, 1)

    tab = jnp.concatenate([gcn_emb[:, :H], gcn_emb[:, H:],
                           gin_emb[:, :H], gin_emb[:, H:]], axis=0)
    gcn0, gin_h = _embg_call()(tab, feat_r)
    gcn_h, no_c, ni_c = _norm_call(dego, degi, gcn0)

    for i in range(L):
        agg = _prop_call()(gcn_h.reshape(2 * NP, H), src_s, dst_s)
        call = _gcn_mid_call if i < L - 1 else _gcn_last_call
        gcn_h = call(agg, ni_c, no_c, gcn_W[i], gcn_b[i].reshape(1, D))

    b1r = mlp_b1.reshape(1, D)
    b2r = mlp_b2.reshape(1, D)
    for i in range(L):
        agg = _prop_call()(gin_h.reshape(2 * NP, H), src_s, dst_s)
        gin_h = _gin_call(gin_h, agg, eps[i].reshape(1, 1),
                          mlp_W1, b1r, mlp_W2, b2r)

    pool, cnt = _pool_call(gcn_h, gin_h, gid_c)
    out = _head_call(pool, cnt,
                     ffnn_W0, ffnn_b0.reshape(1, FF),
                     ffnn_W1, ffnn_b1.reshape(1, FF),
                     fc_W, fc_b.reshape(1, 1))
    return out
